# baseline (speedup 1.0000x reference)
import functools

import jax
import jax.numpy as jnp
import numpy as np
from jax import lax
from jax.experimental import pallas as pl
from jax.experimental.pallas import tpu as pltpu
from jax.experimental.pallas import tpu_sc as plsc

_VOCAB = 100000
_D = 64
_SEQ = 200
_BATCH = 4096
_SCALE = float(np.sqrt(_D))

_NC = 2
_NS = 16
_NW = _NC * _NS

_BB = _BATCH // _NW
_NRING = 4


def _pos_encoding() -> jnp.ndarray:
    position = np.arange(_SEQ)[:, np.newaxis]
    div_term = np.exp(np.arange(0, _D, 2) * -(np.log(10000.0) / _D))
    pe = np.zeros((_SEQ, _D), dtype=np.float32)
    pe[:, 0::2] = np.sin(position * div_term)
    pe[:, 1::2] = np.cos(position * div_term)
    return jnp.asarray(pe)


def _sc_kernel(x_hbm, pe_hbm, w_hbm, out_hbm, xblk_v, pe_v, gbufs, obufs,
               icols, gsems, ssems):
    wid = lax.axis_index("s") * _NC + lax.axis_index("c")

    pltpu.sync_copy(x_hbm.at[pl.ds(wid * _BB, _BB), :], xblk_v)
    pltpu.sync_copy(pe_hbm, pe_v)

    lanes = lax.iota(jnp.int32, 16)

    def prep_and_fire(s, slot):
        scol = jnp.full((16,), s, jnp.int32)
        for j in range(_BB // 16):
            v = plsc.load_gather(xblk_v, [j * 16 + lanes, scol])
            icols[slot][pl.ds(j * 16, 16)] = v
        pltpu.async_copy(w_hbm.at[icols[slot]], gbufs[slot], gsems[slot])

    def drain_gather(slot):
        pltpu.make_async_copy(
            w_hbm.at[pl.ds(0, _BB)], gbufs[slot], gsems[slot]).wait()

    def fire_store(s, slot):
        for t in range(_D // 8):
            pltpu.async_copy(
                obufs[slot].at[pl.ds(t * 8, 8), :],
                out_hbm.at[s, t, wid],
                ssems[slot])

    def drain_store(slot):
        for t in range(_D // 8):
            pltpu.make_async_copy(
                obufs[slot].at[pl.ds(t * 8, 8), :],
                out_hbm.at[0, 0, 0],
                ssems[slot]).wait()

    def compute(s, slot):
        g = gbufs[slot]
        o = obufs[slot]
        for t in range(_D // 16):
            pe16 = pe_v[s, pl.ds(t * 16, 16)]

            def dd_body(dd, _, t=t, pe16=pe16):
                d = t * 16 + dd
                splat = lax.gather(
                    pe16, jnp.full((16, 1), dd, jnp.int32),
                    dimension_numbers=lax.GatherDimensionNumbers(
                        offset_dims=(), collapsed_slice_dims=(0,),
                        start_index_map=(0,)),
                    slice_sizes=(1,),
                    mode=lax.GatherScatterMode.PROMISE_IN_BOUNDS)
                dcol = jnp.full((16,), d, jnp.int32)
                for j in range(_BB // 16):
                    v = plsc.load_gather(g, [j * 16 + lanes, dcol])
                    o[d, pl.ds(j * 16, 16)] = v * _SCALE + splat
                return _

            lax.fori_loop(0, 16, dd_body, 0)

    for q in range(_NRING - 1):
        prep_and_fire(q, q)

    def outer_body(i, _):
        for b in range(_NRING):
            q = i * _NRING + b
            drain_gather(b)
            nslot = (b + _NRING - 1) % _NRING

            @pl.when(q + _NRING - 1 < _SEQ)
            def _refill():
                @pl.when(q >= 1)
                def _wait_prev_store():
                    drain_store(nslot)
                prep_and_fire(q + _NRING - 1, nslot)

            compute(q, b)
            fire_store(q, b)
        return _

    lax.fori_loop(0, _SEQ // _NRING, outer_body, 0)

    for b in range(_NRING):
        drain_store(b)


@jax.jit
def _run(x, pe, W):
    mesh = plsc.VectorSubcoreMesh(core_axis_name="c", subcore_axis_name="s")
    f = functools.partial(
        pl.kernel,
        mesh=mesh,
        out_type=jax.ShapeDtypeStruct((_SEQ, _D // 8, _NW, 8, _BB),
                                      jnp.float32),
        scratch_types=[
            pltpu.VMEM((_BB, _SEQ), jnp.int32),
            pltpu.VMEM((_SEQ, _D), jnp.float32),
            [pltpu.VMEM((_BB, _D), jnp.float32)] * _NRING,
            [pltpu.VMEM((_D, _BB), jnp.float32)] * _NRING,
            [pltpu.VMEM((_BB,), jnp.int32)] * _NRING,
            [pltpu.SemaphoreType.DMA] * _NRING,
            [pltpu.SemaphoreType.DMA] * _NRING,
        ],
        compiler_params=pltpu.CompilerParams(
            use_tc_tiling_on_sc=False, needs_layout_passes=False),
    )(_sc_kernel)
    return f(x, pe, W)


def kernel(x, W):
    phys = _run(x, _pos_encoding(), W)
    return phys.transpose(2, 4, 0, 1, 3).reshape(_BATCH, _SEQ, _D)
